# baseline (speedup 1.0000x reference)
import functools

import jax
import jax.numpy as jnp
from jax import lax
from jax.experimental import pallas as pl
from jax.experimental.pallas import tpu as pltpu
from jax.experimental.pallas import tpu_sc as plsc

D = 32
NC, NS, L = 2, 16, 16
NW = NC * NS
CHUNK = 128
NCH = 4
BPW = NCH * CHUNK
BATCH = NW * BPW

_mesh = plsc.VectorSubcoreMesh(core_axis_name="c", subcore_axis_name="s")


def _gather_body(uidx, pidx, nidx, utab, itab, u_hbm, p_hbm, n_hbm,
                 uidx_v, pidx_v, nidx_v, b0, b1, s0, s1):
    bufs = (b0, b1)
    sems = (s0, s1)
    wid = lax.axis_index("s") * NC + lax.axis_index("c")
    base = wid * NCH
    pltpu.sync_copy(uidx.at[pl.ds(base, NCH)], uidx_v)
    pltpu.sync_copy(pidx.at[pl.ds(base, NCH)], pidx_v)
    pltpu.sync_copy(nidx.at[pl.ds(base, NCH)], nidx_v)

    def fire(j):
        b = bufs[j % 2]
        s = sems[j % 2]
        return (
            pltpu.async_copy(utab.at[uidx_v.at[j]], b.at[0], s),
            pltpu.async_copy(itab.at[pidx_v.at[j]], b.at[1], s),
            pltpu.async_copy(itab.at[nidx_v.at[j]], b.at[2], s),
        )

    descs = {0: fire(0)}
    for j in range(NCH):
        if j + 1 < NCH:
            descs[j + 1] = fire(j + 1)
        for dsc in descs[j]:
            dsc.wait()
        b = bufs[j % 2]
        out = pl.ds(wid * BPW + j * CHUNK, CHUNK)
        pltpu.sync_copy(b.at[0], u_hbm.at[out])
        pltpu.sync_copy(b.at[1], p_hbm.at[out])
        pltpu.sync_copy(b.at[2], n_hbm.at[out])


_gather_kernel = functools.partial(
    pl.kernel,
    mesh=_mesh,
    compiler_params=pltpu.CompilerParams(use_tc_tiling_on_sc=False),
    out_type=(jax.ShapeDtypeStruct((BATCH, D), jnp.float32),
              jax.ShapeDtypeStruct((BATCH, D), jnp.float32),
              jax.ShapeDtypeStruct((BATCH, D), jnp.float32)),
    scratch_types=[
        pltpu.VMEM((NCH, CHUNK), jnp.int32),
        pltpu.VMEM((NCH, CHUNK), jnp.int32),
        pltpu.VMEM((NCH, CHUNK), jnp.int32),
        pltpu.VMEM((3, CHUNK, D), jnp.float32),
        pltpu.VMEM((3, CHUNK, D), jnp.float32),
        pltpu.SemaphoreType.DMA,
        pltpu.SemaphoreType.DMA,
    ],
)(_gather_body)


_DOT_ROWS = 2048


def _dot_body(u_ref, p_ref, n_ref, pos_ref, neg_ref):
    u = u_ref[...]
    ones = jnp.ones((D, 1), jnp.float32)
    pos_ref[...] = lax.dot(u * p_ref[...], ones,
                           precision=lax.Precision.HIGHEST)
    neg_ref[...] = lax.dot(u * n_ref[...], ones,
                           precision=lax.Precision.HIGHEST)


def _dot(urows, prows, nrows):
    grid = BATCH // _DOT_ROWS
    return pl.pallas_call(
        _dot_body,
        grid=(grid,),
        in_specs=[pl.BlockSpec((_DOT_ROWS, D), lambda i: (i, 0))] * 3,
        out_specs=[pl.BlockSpec((_DOT_ROWS, 1), lambda i: (i, 0))] * 2,
        out_shape=[jax.ShapeDtypeStruct((BATCH, 1), jnp.float32)] * 2,
    )(urows, prows, nrows)


def kernel(user_indices, pos_item_indices, neg_item_indices, user_table, item_table):
    u2 = user_indices.astype(jnp.int32).reshape(NW * NCH, CHUNK)
    p2 = pos_item_indices.astype(jnp.int32).reshape(NW * NCH, CHUNK)
    n2 = neg_item_indices.astype(jnp.int32).reshape(NW * NCH, CHUNK)
    urows, prows, nrows = _gather_kernel(u2, p2, n2, user_table, item_table)
    pos, neg = _dot(urows, prows, nrows)
    return pos.reshape(BATCH), neg.reshape(BATCH)


# speedup vs baseline: 1.4716x; 1.4716x over previous
import functools

import jax
import jax.numpy as jnp
from jax import lax
from jax.experimental import pallas as pl
from jax.experimental.pallas import tpu as pltpu
from jax.experimental.pallas import tpu_sc as plsc

D = 32
NC, NS, L = 2, 16, 16
NW = NC * NS
CHUNK = 128
BPW = 512
BATCH = NW * BPW

_mesh = plsc.VectorSubcoreMesh(core_axis_name="c", subcore_axis_name="s")

NCH = 4


def _gather_body(uidx, pidx, nidx, utab, itab, u_hbm, p_hbm, n_hbm,
                 uidx_v, pidx_v, nidx_v, ub0, pb0, nb0, ub1, pb1, nb1,
                 s0, s1):
    bufs = ((ub0, pb0, nb0), (ub1, pb1, nb1))
    sems = (s0, s1)
    wid = lax.axis_index("s") * NC + lax.axis_index("c")
    base8 = wid * 8
    pltpu.sync_copy(uidx.at[pl.ds(base8, 4)], uidx_v)
    pltpu.sync_copy(pidx.at[pl.ds(base8, 4)], pidx_v)
    pltpu.sync_copy(nidx.at[pl.ds(base8, 4)], nidx_v)

    def fire(j):
        b = bufs[j % 2]
        s = sems[j % 2]

        def issue(g, carry):
            c = g * L
            uv = uidx_v[j, pl.ds(c, L)]
            pv = pidx_v[j, pl.ds(c, L)]
            nv = nidx_v[j, pl.ds(c, L)]
            for k in range(L):
                i = c + k
                pltpu.async_copy(utab.at[pl.ds(uv[k], 1)],
                                 b[0].at[pl.ds(i, 1)], s)
                pltpu.async_copy(itab.at[pl.ds(pv[k], 1)],
                                 b[1].at[pl.ds(i, 1)], s)
                pltpu.async_copy(itab.at[pl.ds(nv[k], 1)],
                                 b[2].at[pl.ds(i, 1)], s)
            return carry

        lax.fori_loop(0, CHUNK // L, issue, 0)

    fire(0)
    for j in range(NCH):
        if j + 1 < NCH:
            fire(j + 1)
        b = bufs[j % 2]
        s = sems[j % 2]

        def drain(i, carry, b=b, s=s):
            pltpu.make_async_copy(utab.at[pl.ds(0, 1)],
                                  b[0].at[pl.ds(0, 1)], s).wait()
            pltpu.make_async_copy(itab.at[pl.ds(0, 1)],
                                  b[1].at[pl.ds(0, 1)], s).wait()
            pltpu.make_async_copy(itab.at[pl.ds(0, 1)],
                                  b[2].at[pl.ds(0, 1)], s).wait()
            return carry

        lax.fori_loop(0, CHUNK, drain, 0)
        out = pl.ds(wid * BPW + j * CHUNK, CHUNK)
        pltpu.sync_copy(b[0], u_hbm.at[out])
        pltpu.sync_copy(b[1], p_hbm.at[out])
        pltpu.sync_copy(b[2], n_hbm.at[out])


_gather_kernel = functools.partial(
    pl.kernel,
    mesh=_mesh,
    compiler_params=pltpu.CompilerParams(use_tc_tiling_on_sc=True),
    out_type=(jax.ShapeDtypeStruct((BATCH, D), jnp.float32),
              jax.ShapeDtypeStruct((BATCH, D), jnp.float32),
              jax.ShapeDtypeStruct((BATCH, D), jnp.float32)),
    scratch_types=[
        pltpu.VMEM((4, CHUNK), jnp.int32),
        pltpu.VMEM((4, CHUNK), jnp.int32),
        pltpu.VMEM((4, CHUNK), jnp.int32),
        pltpu.VMEM((CHUNK, D), jnp.float32),
        pltpu.VMEM((CHUNK, D), jnp.float32),
        pltpu.VMEM((CHUNK, D), jnp.float32),
        pltpu.VMEM((CHUNK, D), jnp.float32),
        pltpu.VMEM((CHUNK, D), jnp.float32),
        pltpu.VMEM((CHUNK, D), jnp.float32),
        pltpu.SemaphoreType.DMA,
        pltpu.SemaphoreType.DMA,
    ],
)(_gather_body)


_DOT_ROWS = 2048


def _dot_body(u_ref, p_ref, n_ref, pos_ref, neg_ref):
    u = u_ref[...]
    ones = jnp.ones((D, 1), jnp.float32)
    pos_ref[...] = lax.dot(u * p_ref[...], ones,
                           precision=lax.Precision.HIGHEST)
    neg_ref[...] = lax.dot(u * n_ref[...], ones,
                           precision=lax.Precision.HIGHEST)


def _dot(urows, prows, nrows):
    grid = BATCH // _DOT_ROWS
    return pl.pallas_call(
        _dot_body,
        grid=(grid,),
        in_specs=[pl.BlockSpec((_DOT_ROWS, D), lambda i: (i, 0))] * 3,
        out_specs=[pl.BlockSpec((_DOT_ROWS, 1), lambda i: (i, 0))] * 2,
        out_shape=[jax.ShapeDtypeStruct((BATCH, 1), jnp.float32)] * 2,
    )(urows, prows, nrows)


def _pad_idx(x):
    return jnp.pad(x.reshape(NW, 4, CHUNK), ((0, 0), (0, 4), (0, 0))).reshape(
        NW * 8, CHUNK)


def kernel(user_indices, pos_item_indices, neg_item_indices, user_table, item_table):
    u2 = _pad_idx(user_indices.astype(jnp.int32))
    p2 = _pad_idx(pos_item_indices.astype(jnp.int32))
    n2 = _pad_idx(neg_item_indices.astype(jnp.int32))
    urows, prows, nrows = _gather_kernel(u2, p2, n2, user_table, item_table)
    pos, neg = _dot(urows, prows, nrows)
    return pos.reshape(BATCH), neg.reshape(BATCH)
